# Initial kernel scaffold; baseline (speedup 1.0000x reference)
import functools

import jax
import jax.numpy as jnp
from jax import lax
from jax.experimental import pallas as pl
from jax.experimental.pallas import tpu as pltpu
from jax.experimental.pallas import tpu_sc as plsc

N = 10000
E = 320000
H = 128
STEPS = 6
NG = 64
NC = 2
MLP_H = 2 * H

SC_CORES = 2
SC_SUBCORES = 16
NWORK = SC_CORES * SC_SUBCORES
EDGES_PER_WORKER = E // NWORK
CHUNK = 80
NCHUNK = EDGES_PER_WORKER // CHUNK
ROWS_PER_SUB = N // SC_SUBCORES
ZROWS = 125


_sc_mesh = plsc.VectorSubcoreMesh(core_axis_name="c", subcore_axis_name="s")


@functools.partial(
    pl.kernel,
    mesh=_sc_mesh,
    out_type=jax.ShapeDtypeStruct((SC_CORES, N, H), jnp.float32),
    scratch_types=[
        pltpu.VMEM((CHUNK,), jnp.int32),
        pltpu.VMEM((CHUNK,), jnp.int32),
        pltpu.VMEM((CHUNK, H), jnp.float32),
        pltpu.VMEM((ZROWS, H), jnp.float32),
        pltpu.VMEM_SHARED((N, H), jnp.float32),
        pltpu.SemaphoreType.DMA,
    ],
)
def _sc_segment_sum(m_hbm, src_hbm, dst_hbm, out_hbm,
                    src_v, dst_v, rows_v, zero_v, agg_sh, sem):
    c = lax.axis_index("c")
    s = lax.axis_index("s")
    wid = c * SC_SUBCORES + s

    @pl.loop(0, ZROWS)
    def _zero_rows(r):
        @pl.loop(0, H, step=16)
        def _zero_lanes(col):
            zero_v[r, pl.ds(col, 16)] = jnp.zeros((16,), jnp.float32)

    row0 = s * ROWS_PER_SUB

    @pl.loop(0, ROWS_PER_SUB // ZROWS)
    def _zero_spmem(t):
        pltpu.sync_copy(zero_v, agg_sh.at[pl.ds(row0 + t * ZROWS, ZROWS)])

    plsc.subcore_barrier()

    @pl.loop(0, NCHUNK)
    def _edges(j):
        base = wid * EDGES_PER_WORKER + j * CHUNK
        pltpu.sync_copy(src_hbm.at[pl.ds(base, CHUNK)], src_v)
        pltpu.sync_copy(dst_hbm.at[pl.ds(base, CHUNK)], dst_v)
        pltpu.async_copy(m_hbm.at[src_v], rows_v, sem).wait()
        pltpu.sync_copy(rows_v, agg_sh.at[dst_v], add=True)

    plsc.subcore_barrier()

    pltpu.sync_copy(agg_sh.at[pl.ds(row0, ROWS_PER_SUB)],
                    out_hbm.at[c, pl.ds(row0, ROWS_PER_SUB)])


_MM_ROWS = 2000


def _mm_body(h_ref, w_ref, o_ref):
    o_ref[...] = jnp.dot(h_ref[...], w_ref[...],
                         preferred_element_type=jnp.float32)


def _tc_matmul(h, w):
    return pl.pallas_call(
        _mm_body,
        grid=(N // _MM_ROWS,),
        in_specs=[
            pl.BlockSpec((_MM_ROWS, H), lambda i: (i, 0)),
            pl.BlockSpec((H, H), lambda i: (0, 0)),
        ],
        out_specs=pl.BlockSpec((_MM_ROWS, H), lambda i: (i, 0)),
        out_shape=jax.ShapeDtypeStruct((N, H), jnp.float32),
    )(h, w)


_GRU_ROWS = 2000


def _gru_body(parts_ref, h_ref, w_ih_ref, w_hh_ref, b_ih_ref, b_hh_ref, o_ref):
    agg = parts_ref[0] + parts_ref[1]
    h = h_ref[...]
    gi = lax.dot_general(agg, w_ih_ref[...],
                         (((1,), (1,)), ((), ())),
                         preferred_element_type=jnp.float32) + b_ih_ref[...]
    gh = lax.dot_general(h, w_hh_ref[...],
                         (((1,), (1,)), ((), ())),
                         preferred_element_type=jnp.float32) + b_hh_ref[...]
    i_r, i_z, i_n = gi[:, :H], gi[:, H:2 * H], gi[:, 2 * H:]
    h_r, h_z, h_n = gh[:, :H], gh[:, H:2 * H], gh[:, 2 * H:]
    r = jax.nn.sigmoid(i_r + h_r)
    z = jax.nn.sigmoid(i_z + h_z)
    n = jnp.tanh(i_n + r * h_n)
    o_ref[...] = (1.0 - z) * n + z * h


def _tc_gru(parts, h, w_ih, w_hh, b_ih2, b_hh2):
    return pl.pallas_call(
        _gru_body,
        grid=(N // _GRU_ROWS,),
        in_specs=[
            pl.BlockSpec((SC_CORES, _GRU_ROWS, H), lambda i: (0, i, 0)),
            pl.BlockSpec((_GRU_ROWS, H), lambda i: (i, 0)),
            pl.BlockSpec((3 * H, H), lambda i: (0, 0)),
            pl.BlockSpec((3 * H, H), lambda i: (0, 0)),
            pl.BlockSpec((1, 3 * H), lambda i: (0, 0)),
            pl.BlockSpec((1, 3 * H), lambda i: (0, 0)),
        ],
        out_specs=pl.BlockSpec((_GRU_ROWS, H), lambda i: (i, 0)),
        out_shape=jax.ShapeDtypeStruct((N, H), jnp.float32),
    )(parts, h, w_ih, w_hh, b_ih2, b_hh2)


_POOL_ROWS = 1000
_POOL_BLOCKS = N // _POOL_ROWS


def _pool_mlp_body(h_ref, batch_ref, w1_ref, b1_ref, w2_ref, b2_ref,
                   w3_ref, b3_ref, wc_ref, bc_ref, o_ref, pool_acc):
    i = pl.program_id(0)

    @pl.when(i == 0)
    def _():
        pool_acc[...] = jnp.zeros((NG, H), jnp.float32)

    bat = batch_ref[0, 0, :]
    gids = lax.broadcasted_iota(jnp.int32, (NG, _POOL_ROWS), 0)
    onehot = (bat[None, :] == gids).astype(jnp.float32)
    pool_acc[...] += jnp.dot(onehot, h_ref[...],
                             preferred_element_type=jnp.float32)

    @pl.when(i == _POOL_BLOCKS - 1)
    def _():
        g = pool_acc[...]
        f = jax.nn.relu(lax.dot_general(g, w1_ref[...],
                                        (((1,), (1,)), ((), ())),
                                        preferred_element_type=jnp.float32)
                        + b1_ref[...])
        f = jax.nn.relu(lax.dot_general(f, w2_ref[...],
                                        (((1,), (1,)), ((), ())),
                                        preferred_element_type=jnp.float32)
                        + b2_ref[...])
        f = jax.nn.relu(lax.dot_general(f, w3_ref[...],
                                        (((1,), (1,)), ((), ())),
                                        preferred_element_type=jnp.float32)
                        + b3_ref[...])
        o_ref[...] = lax.dot_general(f, wc_ref[...],
                                     (((1,), (1,)), ((), ())),
                                     preferred_element_type=jnp.float32) \
            + bc_ref[...]


def _tc_pool_mlp(h, batch3, W1, b1_2, W2, b2_2, W3, b3_2, Wc, bc_2):
    def full(shape):
        return pl.BlockSpec(shape, lambda i: tuple(0 for _ in shape))
    return pl.pallas_call(
        _pool_mlp_body,
        grid=(_POOL_BLOCKS,),
        in_specs=[
            pl.BlockSpec((_POOL_ROWS, H), lambda i: (i, 0)),
            pl.BlockSpec((1, 1, _POOL_ROWS), lambda i: (i, 0, 0)),
            full((MLP_H, H)),
            full((1, MLP_H)),
            full((H, MLP_H)),
            full((1, H)),
            full((MLP_H, H)),
            full((1, MLP_H)),
            full((NC, MLP_H)),
            full((1, NC)),
        ],
        out_specs=pl.BlockSpec((NG, NC), lambda i: (0, 0)),
        out_shape=jax.ShapeDtypeStruct((NG, NC), jnp.float32),
        scratch_shapes=[pltpu.VMEM((NG, H), jnp.float32)],
    )(h, batch3, W1, b1_2, W2, b2_2, W3, b3_2, Wc, bc_2)


def kernel(x, edge_index, batch, ggnn_weight, w_ih, w_hh, b_ih, b_hh,
           W1, b1, W2, b2, W3, b3, Wc, bc):
    src = edge_index[0]
    dst = edge_index[1]
    b_ih2 = b_ih.reshape(1, 3 * H)
    b_hh2 = b_hh.reshape(1, 3 * H)
    batch3 = batch.reshape(_POOL_BLOCKS, 1, _POOL_ROWS)

    h = x
    for i in range(STEPS):
        m = _tc_matmul(h, ggnn_weight[i])
        parts = _sc_segment_sum(m, src, dst)
        h = _tc_gru(parts, h, w_ih, w_hh, b_ih2, b_hh2)

    return _tc_pool_mlp(h, batch3, W1, b1.reshape(1, MLP_H),
                        W2, b2.reshape(1, H), W3, b3.reshape(1, MLP_H),
                        Wc, bc.reshape(1, NC))


# speedup vs baseline: 4.4746x; 4.4746x over previous
import functools

import jax
import jax.numpy as jnp
from jax import lax
from jax.experimental import pallas as pl
from jax.experimental.pallas import tpu as pltpu
from jax.experimental.pallas import tpu_sc as plsc

N = 10000
E = 320000
H = 128
STEPS = 6
NG = 64
NC = 2
MLP_H = 2 * H

SC_CORES = 2
SC_SUBCORES = 16
NWORK = SC_CORES * SC_SUBCORES
EDGES_PER_WORKER = E // NWORK
CHUNK = 80
NCHUNK = EDGES_PER_WORKER // CHUNK
ZROWS = 80
NRBLK = N // ZROWS


def _sc_segment_sum_body(m_hbm, src_hbm, dst_hbm, out_hbm,
                         src_v, dst_v, rows_v, zero_v, agg_sh, sem):
    c = lax.axis_index("c")
    s = lax.axis_index("s")
    wid = c * SC_SUBCORES + s

    @pl.loop(0, ZROWS)
    def _zero_rows(r):
        @pl.loop(0, H, step=16)
        def _zero_lanes(col):
            zero_v[r, pl.ds(col, 16)] = jnp.zeros((16,), jnp.float32)

    @pl.loop(s, NRBLK, step=SC_SUBCORES)
    def _zero_spmem(t):
        pltpu.sync_copy(zero_v, agg_sh.at[pl.ds(t * ZROWS, ZROWS)])

    plsc.subcore_barrier()

    @pl.loop(0, NCHUNK)
    def _edges(j):
        base = wid * EDGES_PER_WORKER + j * CHUNK
        pltpu.sync_copy(src_hbm.at[pl.ds(base, CHUNK)], src_v)
        pltpu.sync_copy(dst_hbm.at[pl.ds(base, CHUNK)], dst_v)
        pltpu.async_copy(m_hbm.at[src_v], rows_v, sem).wait()
        pltpu.sync_copy(rows_v, agg_sh.at[dst_v], add=True)

    plsc.subcore_barrier()

    @pl.loop(s, NRBLK, step=SC_SUBCORES)
    def _dump(t):
        pltpu.sync_copy(agg_sh.at[pl.ds(t * ZROWS, ZROWS)],
                        out_hbm.at[c, pl.ds(t * ZROWS, ZROWS)])


@functools.lru_cache(maxsize=1)
def _get_sc_segment_sum():
    mesh = plsc.VectorSubcoreMesh(core_axis_name="c", subcore_axis_name="s")
    return pl.kernel(
        _sc_segment_sum_body,
        mesh=mesh,
        out_type=jax.ShapeDtypeStruct((SC_CORES, N, H), jnp.float32),
        scratch_types=[
            pltpu.VMEM((CHUNK,), jnp.int32),
            pltpu.VMEM((CHUNK,), jnp.int32),
            pltpu.VMEM((CHUNK, H), jnp.float32),
            pltpu.VMEM((ZROWS, H), jnp.float32),
            pltpu.VMEM_SHARED((N, H), jnp.float32),
            pltpu.SemaphoreType.DMA,
        ],
    )


_MM_ROWS = 2000


def _mm_body(h_ref, w_ref, o_ref):
    o_ref[...] = jnp.dot(h_ref[...], w_ref[...],
                         preferred_element_type=jnp.float32)


def _tc_matmul(h, w):
    return pl.pallas_call(
        _mm_body,
        grid=(N // _MM_ROWS,),
        in_specs=[
            pl.BlockSpec((_MM_ROWS, H), lambda i: (i, 0)),
            pl.BlockSpec((H, H), lambda i: (0, 0)),
        ],
        out_specs=pl.BlockSpec((_MM_ROWS, H), lambda i: (i, 0)),
        out_shape=jax.ShapeDtypeStruct((N, H), jnp.float32),
    )(h, w)


_GRU_ROWS = 2000


def _gru_body(parts_ref, h_ref, w_ih_ref, w_hh_ref, b_ih_ref, b_hh_ref, o_ref):
    agg = parts_ref[0] + parts_ref[1]
    h = h_ref[...]
    gi = lax.dot_general(agg, w_ih_ref[...],
                         (((1,), (1,)), ((), ())),
                         preferred_element_type=jnp.float32) + b_ih_ref[...]
    gh = lax.dot_general(h, w_hh_ref[...],
                         (((1,), (1,)), ((), ())),
                         preferred_element_type=jnp.float32) + b_hh_ref[...]
    i_r, i_z, i_n = gi[:, :H], gi[:, H:2 * H], gi[:, 2 * H:]
    h_r, h_z, h_n = gh[:, :H], gh[:, H:2 * H], gh[:, 2 * H:]
    r = jax.nn.sigmoid(i_r + h_r)
    z = jax.nn.sigmoid(i_z + h_z)
    n = jnp.tanh(i_n + r * h_n)
    o_ref[...] = (1.0 - z) * n + z * h


def _tc_gru(parts, h, w_ih, w_hh, b_ih2, b_hh2):
    return pl.pallas_call(
        _gru_body,
        grid=(N // _GRU_ROWS,),
        in_specs=[
            pl.BlockSpec((SC_CORES, _GRU_ROWS, H), lambda i: (0, i, 0)),
            pl.BlockSpec((_GRU_ROWS, H), lambda i: (i, 0)),
            pl.BlockSpec((3 * H, H), lambda i: (0, 0)),
            pl.BlockSpec((3 * H, H), lambda i: (0, 0)),
            pl.BlockSpec((1, 3 * H), lambda i: (0, 0)),
            pl.BlockSpec((1, 3 * H), lambda i: (0, 0)),
        ],
        out_specs=pl.BlockSpec((_GRU_ROWS, H), lambda i: (i, 0)),
        out_shape=jax.ShapeDtypeStruct((N, H), jnp.float32),
    )(parts, h, w_ih, w_hh, b_ih2, b_hh2)


_POOL_ROWS = 1000
_POOL_BLOCKS = N // _POOL_ROWS


def _pool_mlp_body(h_ref, batch_ref, w1_ref, b1_ref, w2_ref, b2_ref,
                   w3_ref, b3_ref, wc_ref, bc_ref, o_ref, pool_acc):
    i = pl.program_id(0)

    @pl.when(i == 0)
    def _():
        pool_acc[...] = jnp.zeros((NG, H), jnp.float32)

    bat = batch_ref[0, 0, :]
    gids = lax.broadcasted_iota(jnp.int32, (NG, _POOL_ROWS), 0)
    onehot = (bat[None, :] == gids).astype(jnp.float32)
    pool_acc[...] += jnp.dot(onehot, h_ref[...],
                             preferred_element_type=jnp.float32)

    @pl.when(i == _POOL_BLOCKS - 1)
    def _():
        g = pool_acc[...]
        f = jax.nn.relu(lax.dot_general(g, w1_ref[...],
                                        (((1,), (1,)), ((), ())),
                                        preferred_element_type=jnp.float32)
                        + b1_ref[...])
        f = jax.nn.relu(lax.dot_general(f, w2_ref[...],
                                        (((1,), (1,)), ((), ())),
                                        preferred_element_type=jnp.float32)
                        + b2_ref[...])
        f = jax.nn.relu(lax.dot_general(f, w3_ref[...],
                                        (((1,), (1,)), ((), ())),
                                        preferred_element_type=jnp.float32)
                        + b3_ref[...])
        o_ref[...] = lax.dot_general(f, wc_ref[...],
                                     (((1,), (1,)), ((), ())),
                                     preferred_element_type=jnp.float32) \
            + bc_ref[...]


def _tc_pool_mlp(h, batch3, W1, b1_2, W2, b2_2, W3, b3_2, Wc, bc_2):
    def full(shape):
        return pl.BlockSpec(shape, lambda i: tuple(0 for _ in shape))
    return pl.pallas_call(
        _pool_mlp_body,
        grid=(_POOL_BLOCKS,),
        in_specs=[
            pl.BlockSpec((_POOL_ROWS, H), lambda i: (i, 0)),
            pl.BlockSpec((1, 1, _POOL_ROWS), lambda i: (i, 0, 0)),
            full((MLP_H, H)),
            full((1, MLP_H)),
            full((H, MLP_H)),
            full((1, H)),
            full((MLP_H, H)),
            full((1, MLP_H)),
            full((NC, MLP_H)),
            full((1, NC)),
        ],
        out_specs=pl.BlockSpec((NG, NC), lambda i: (0, 0)),
        out_shape=jax.ShapeDtypeStruct((NG, NC), jnp.float32),
        scratch_shapes=[pltpu.VMEM((NG, H), jnp.float32)],
    )(h, batch3, W1, b1_2, W2, b2_2, W3, b3_2, Wc, bc_2)


def kernel(x, edge_index, batch, ggnn_weight, w_ih, w_hh, b_ih, b_hh,
           W1, b1, W2, b2, W3, b3, Wc, bc):
    src = edge_index[0]
    dst = edge_index[1]
    b_ih2 = b_ih.reshape(1, 3 * H)
    b_hh2 = b_hh.reshape(1, 3 * H)
    batch3 = batch.reshape(_POOL_BLOCKS, 1, _POOL_ROWS)

    h = x
    for i in range(STEPS):
        m = _tc_matmul(h, ggnn_weight[i])
        parts = _get_sc_segment_sum()(m, src, dst)
        h = _tc_gru(parts, h, w_ih, w_hh, b_ih2, b_hh2)

    return _tc_pool_mlp(h, batch3, W1, b1.reshape(1, MLP_H),
                        W2, b2.reshape(1, H), W3, b3.reshape(1, MLP_H),
                        Wc, bc.reshape(1, NC))


# speedup vs baseline: 11.4446x; 2.5577x over previous
import functools

import jax
import jax.numpy as jnp
from jax import lax
from jax.experimental import pallas as pl
from jax.experimental.pallas import tpu as pltpu
from jax.experimental.pallas import tpu_sc as plsc

N = 10000
E = 320000
H = 128
STEPS = 6
NG = 64
NC = 2
MLP_H = 2 * H

SC_CORES = 2
SC_SUBCORES = 16
NWORK = SC_CORES * SC_SUBCORES
CHUNK = 128
TOT_CH = E // CHUNK
CH_PER_W = TOT_CH // NWORK
EXTRA_CH = TOT_CH - CH_PER_W * NWORK
ZBLK = CHUNK
NZB = N // ZBLK
ZTAIL = N - NZB * ZBLK


def _sc_segment_sum_body(m_hbm, src_hbm, dst_hbm, out_hbm,
                         ia, ja, ib, jb, rows_a, rows_b, agg_sh,
                         sem_ia, sem_ja, sem_ib, sem_jb, sem_a, sem_b):
    c = lax.axis_index("c")
    s = lax.axis_index("s")
    wid = c * SC_SUBCORES + s
    ch0 = wid * CH_PER_W

    def _load_i(j, buf, sem):
        pltpu.async_copy(src_hbm.at[pl.ds(j * CHUNK, CHUNK)], buf, sem)

    def _load_j(j, buf, sem):
        pltpu.async_copy(dst_hbm.at[pl.ds(j * CHUNK, CHUNK)], buf, sem)

    def _iwait(buf, sem):
        pltpu.make_async_copy(src_hbm.at[pl.ds(0, CHUNK)], buf, sem).wait()

    def _gather(ibuf, buf, sem):
        pltpu.async_copy(m_hbm.at[ibuf], buf, sem)

    def _gwait(buf, sem):
        pltpu.make_async_copy(m_hbm.at[ia], buf, sem).wait()

    def _scat(buf, jbuf):
        pltpu.sync_copy(buf, agg_sh.at[jbuf], add=True)

    _load_i(ch0, ia, sem_ia)
    _load_j(ch0, ja, sem_ja)
    _load_i(ch0 + 1, ib, sem_ib)
    _load_j(ch0 + 1, jb, sem_jb)

    @pl.loop(0, ZBLK)
    def _zero_rows(r):
        @pl.loop(0, H, step=16)
        def _zero_lanes(col):
            rows_a[r, pl.ds(col, 16)] = jnp.zeros((16,), jnp.float32)

    @pl.loop(s, NZB, step=SC_SUBCORES)
    def _zero_spmem(t):
        pltpu.sync_copy(rows_a, agg_sh.at[pl.ds(t * ZBLK, ZBLK)])

    @pl.when(s == SC_SUBCORES - 1)
    def _zero_tail():
        pltpu.sync_copy(rows_a.at[pl.ds(0, ZTAIL)],
                        agg_sh.at[pl.ds(NZB * ZBLK, ZTAIL)])

    plsc.subcore_barrier()

    _iwait(ia, sem_ia)
    _gather(ia, rows_a, sem_a)
    _iwait(ib, sem_ib)
    _gather(ib, rows_b, sem_b)

    @pl.loop(0, CH_PER_W - 2, step=2)
    def _edges(j):
        _gwait(rows_a, sem_a)
        _load_i(ch0 + j + 2, ia, sem_ia)
        pltpu.make_async_copy(dst_hbm.at[pl.ds(0, CHUNK)], ja, sem_ja).wait()
        _scat(rows_a, ja)
        _load_j(ch0 + j + 2, ja, sem_ja)
        _iwait(ia, sem_ia)
        _gather(ia, rows_a, sem_a)
        _gwait(rows_b, sem_b)
        _load_i(ch0 + j + 3, ib, sem_ib)
        pltpu.make_async_copy(dst_hbm.at[pl.ds(0, CHUNK)], jb, sem_jb).wait()
        _scat(rows_b, jb)
        _load_j(ch0 + j + 3, jb, sem_jb)
        _iwait(ib, sem_ib)
        _gather(ib, rows_b, sem_b)

    _gwait(rows_a, sem_a)
    pltpu.make_async_copy(dst_hbm.at[pl.ds(0, CHUNK)], ja, sem_ja).wait()
    _scat(rows_a, ja)
    _gwait(rows_b, sem_b)
    pltpu.make_async_copy(dst_hbm.at[pl.ds(0, CHUNK)], jb, sem_jb).wait()
    _scat(rows_b, jb)

    @pl.when(wid < EXTRA_CH)
    def _extra():
        e = NWORK * CH_PER_W + wid
        pltpu.sync_copy(src_hbm.at[pl.ds(e * CHUNK, CHUNK)], ia)
        pltpu.sync_copy(dst_hbm.at[pl.ds(e * CHUNK, CHUNK)], ja)
        pltpu.async_copy(m_hbm.at[ia], rows_a, sem_a).wait()
        _scat(rows_a, ja)

    plsc.subcore_barrier()

    @pl.loop(s, NZB, step=SC_SUBCORES)
    def _dump(t):
        pltpu.sync_copy(agg_sh.at[pl.ds(t * ZBLK, ZBLK)],
                        out_hbm.at[c, pl.ds(t * ZBLK, ZBLK)])

    @pl.when(s == SC_SUBCORES - 1)
    def _dump_tail():
        pltpu.sync_copy(agg_sh.at[pl.ds(NZB * ZBLK, ZTAIL)],
                        out_hbm.at[c, pl.ds(NZB * ZBLK, ZTAIL)])


@functools.lru_cache(maxsize=1)
def _get_sc_segment_sum():
    mesh = plsc.VectorSubcoreMesh(core_axis_name="c", subcore_axis_name="s")
    return pl.kernel(
        _sc_segment_sum_body,
        mesh=mesh,
        out_type=jax.ShapeDtypeStruct((SC_CORES, N, H), jnp.float32),
        scratch_types=[
            pltpu.VMEM((CHUNK,), jnp.int32),
            pltpu.VMEM((CHUNK,), jnp.int32),
            pltpu.VMEM((CHUNK,), jnp.int32),
            pltpu.VMEM((CHUNK,), jnp.int32),
            pltpu.VMEM((CHUNK, H), jnp.float32),
            pltpu.VMEM((CHUNK, H), jnp.float32),
            pltpu.VMEM_SHARED((N, H), jnp.float32),
            pltpu.SemaphoreType.DMA,
            pltpu.SemaphoreType.DMA,
            pltpu.SemaphoreType.DMA,
            pltpu.SemaphoreType.DMA,
            pltpu.SemaphoreType.DMA,
            pltpu.SemaphoreType.DMA,
        ],
    )


_MM_ROWS = 2000


def _mm_body(h_ref, w_ref, o_ref):
    o_ref[...] = jnp.dot(h_ref[...], w_ref[...],
                         preferred_element_type=jnp.float32)


def _tc_matmul(h, w):
    return pl.pallas_call(
        _mm_body,
        grid=(N // _MM_ROWS,),
        in_specs=[
            pl.BlockSpec((_MM_ROWS, H), lambda i: (i, 0)),
            pl.BlockSpec((H, H), lambda i: (0, 0)),
        ],
        out_specs=pl.BlockSpec((_MM_ROWS, H), lambda i: (i, 0)),
        out_shape=jax.ShapeDtypeStruct((N, H), jnp.float32),
    )(h, w)


_GRU_ROWS = 2000


def _gru_body(parts_ref, h_ref, w_ih_ref, w_hh_ref, b_ih_ref, b_hh_ref, o_ref):
    agg = parts_ref[0] + parts_ref[1]
    h = h_ref[...]
    gi = lax.dot_general(agg, w_ih_ref[...],
                         (((1,), (1,)), ((), ())),
                         preferred_element_type=jnp.float32) + b_ih_ref[...]
    gh = lax.dot_general(h, w_hh_ref[...],
                         (((1,), (1,)), ((), ())),
                         preferred_element_type=jnp.float32) + b_hh_ref[...]
    i_r, i_z, i_n = gi[:, :H], gi[:, H:2 * H], gi[:, 2 * H:]
    h_r, h_z, h_n = gh[:, :H], gh[:, H:2 * H], gh[:, 2 * H:]
    r = jax.nn.sigmoid(i_r + h_r)
    z = jax.nn.sigmoid(i_z + h_z)
    n = jnp.tanh(i_n + r * h_n)
    o_ref[...] = (1.0 - z) * n + z * h


def _tc_gru(parts, h, w_ih, w_hh, b_ih2, b_hh2):
    return pl.pallas_call(
        _gru_body,
        grid=(N // _GRU_ROWS,),
        in_specs=[
            pl.BlockSpec((SC_CORES, _GRU_ROWS, H), lambda i: (0, i, 0)),
            pl.BlockSpec((_GRU_ROWS, H), lambda i: (i, 0)),
            pl.BlockSpec((3 * H, H), lambda i: (0, 0)),
            pl.BlockSpec((3 * H, H), lambda i: (0, 0)),
            pl.BlockSpec((1, 3 * H), lambda i: (0, 0)),
            pl.BlockSpec((1, 3 * H), lambda i: (0, 0)),
        ],
        out_specs=pl.BlockSpec((_GRU_ROWS, H), lambda i: (i, 0)),
        out_shape=jax.ShapeDtypeStruct((N, H), jnp.float32),
    )(parts, h, w_ih, w_hh, b_ih2, b_hh2)


_POOL_ROWS = 1000
_POOL_BLOCKS = N // _POOL_ROWS


def _pool_mlp_body(h_ref, batch_ref, w1_ref, b1_ref, w2_ref, b2_ref,
                   w3_ref, b3_ref, wc_ref, bc_ref, o_ref, pool_acc):
    i = pl.program_id(0)

    @pl.when(i == 0)
    def _():
        pool_acc[...] = jnp.zeros((NG, H), jnp.float32)

    bat = batch_ref[0, 0, :]
    gids = lax.broadcasted_iota(jnp.int32, (NG, _POOL_ROWS), 0)
    onehot = (bat[None, :] == gids).astype(jnp.float32)
    pool_acc[...] += jnp.dot(onehot, h_ref[...],
                             preferred_element_type=jnp.float32)

    @pl.when(i == _POOL_BLOCKS - 1)
    def _():
        g = pool_acc[...]
        f = jax.nn.relu(lax.dot_general(g, w1_ref[...],
                                        (((1,), (1,)), ((), ())),
                                        preferred_element_type=jnp.float32)
                        + b1_ref[...])
        f = jax.nn.relu(lax.dot_general(f, w2_ref[...],
                                        (((1,), (1,)), ((), ())),
                                        preferred_element_type=jnp.float32)
                        + b2_ref[...])
        f = jax.nn.relu(lax.dot_general(f, w3_ref[...],
                                        (((1,), (1,)), ((), ())),
                                        preferred_element_type=jnp.float32)
                        + b3_ref[...])
        o_ref[...] = lax.dot_general(f, wc_ref[...],
                                     (((1,), (1,)), ((), ())),
                                     preferred_element_type=jnp.float32) \
            + bc_ref[...]


def _tc_pool_mlp(h, batch3, W1, b1_2, W2, b2_2, W3, b3_2, Wc, bc_2):
    def full(shape):
        return pl.BlockSpec(shape, lambda i: tuple(0 for _ in shape))
    return pl.pallas_call(
        _pool_mlp_body,
        grid=(_POOL_BLOCKS,),
        in_specs=[
            pl.BlockSpec((_POOL_ROWS, H), lambda i: (i, 0)),
            pl.BlockSpec((1, 1, _POOL_ROWS), lambda i: (i, 0, 0)),
            full((MLP_H, H)),
            full((1, MLP_H)),
            full((H, MLP_H)),
            full((1, H)),
            full((MLP_H, H)),
            full((1, MLP_H)),
            full((NC, MLP_H)),
            full((1, NC)),
        ],
        out_specs=pl.BlockSpec((NG, NC), lambda i: (0, 0)),
        out_shape=jax.ShapeDtypeStruct((NG, NC), jnp.float32),
        scratch_shapes=[pltpu.VMEM((NG, H), jnp.float32)],
    )(h, batch3, W1, b1_2, W2, b2_2, W3, b3_2, Wc, bc_2)


def kernel(x, edge_index, batch, ggnn_weight, w_ih, w_hh, b_ih, b_hh,
           W1, b1, W2, b2, W3, b3, Wc, bc):
    src = edge_index[0]
    dst = edge_index[1]
    b_ih2 = b_ih.reshape(1, 3 * H)
    b_hh2 = b_hh.reshape(1, 3 * H)
    batch3 = batch.reshape(_POOL_BLOCKS, 1, _POOL_ROWS)

    h = x
    for i in range(STEPS):
        m = _tc_matmul(h, ggnn_weight[i])
        parts = _get_sc_segment_sum()(m, src, dst)
        h = _tc_gru(parts, h, w_ih, w_hh, b_ih2, b_hh2)

    return _tc_pool_mlp(h, batch3, W1, b1.reshape(1, MLP_H),
                        W2, b2.reshape(1, H), W3, b3.reshape(1, MLP_H),
                        Wc, bc.reshape(1, NC))


# speedup vs baseline: 12.6774x; 1.1077x over previous
import functools

import jax
import jax.numpy as jnp
from jax import lax
from jax.experimental import pallas as pl
from jax.experimental.pallas import tpu as pltpu
from jax.experimental.pallas import tpu_sc as plsc

N = 10000
E = 320000
H = 128
STEPS = 6
NG = 64
NC = 2
MLP_H = 2 * H

SC_CORES = 2
SC_SUBCORES = 16
NWORK = SC_CORES * SC_SUBCORES
CHUNK = 128
TOT_CH = E // CHUNK
CH_PER_W = TOT_CH // NWORK
EXTRA_CH = TOT_CH - CH_PER_W * NWORK
ZBLK = CHUNK
NZB = N // ZBLK
ZTAIL = N - NZB * ZBLK


def _sc_segment_sum_body(m_hbm, src_hbm, dst_hbm, out_hbm,
                         i0, j0, i1, j1, i2, j2, r0, r1, r2, agg_sh,
                         si0, sj0, si1, sj1, si2, sj2,
                         sg0, sg1, sg2, ss0, ss1, ss2):
    c = lax.axis_index("c")
    s = lax.axis_index("s")
    wid = c * SC_SUBCORES + s
    ch0 = wid * CH_PER_W
    ibufs = (i0, i1, i2)
    jbufs = (j0, j1, j2)
    rbufs = (r0, r1, r2)
    sis = (si0, si1, si2)
    sjs = (sj0, sj1, sj2)
    sgs = (sg0, sg1, sg2)
    sss = (ss0, ss1, ss2)

    def _load_i(j, k):
        pltpu.async_copy(src_hbm.at[pl.ds(j * CHUNK, CHUNK)], ibufs[k], sis[k])

    def _load_j(j, k):
        pltpu.async_copy(dst_hbm.at[pl.ds(j * CHUNK, CHUNK)], jbufs[k], sjs[k])

    def _iwait(k):
        pltpu.make_async_copy(src_hbm.at[pl.ds(0, CHUNK)], ibufs[k],
                              sis[k]).wait()

    def _jwait(k):
        pltpu.make_async_copy(dst_hbm.at[pl.ds(0, CHUNK)], jbufs[k],
                              sjs[k]).wait()

    def _gather(k):
        pltpu.async_copy(m_hbm.at[ibufs[k]], rbufs[k], sgs[k])

    def _gwait(k):
        pltpu.make_async_copy(m_hbm.at[ibufs[k]], rbufs[k], sgs[k]).wait()

    def _scat(k):
        pltpu.async_copy(rbufs[k], agg_sh.at[jbufs[k]], sss[k], add=True)

    def _swait(k):
        pltpu.make_async_copy(rbufs[k], agg_sh.at[jbufs[k]], sss[k]).wait()

    for k in range(3):
        _load_i(ch0 + k, k)
        _load_j(ch0 + k, k)

    @pl.loop(0, ZBLK)
    def _zero_rows(r):
        @pl.loop(0, H, step=16)
        def _zero_lanes(col):
            r0[r, pl.ds(col, 16)] = jnp.zeros((16,), jnp.float32)

    @pl.loop(s, NZB, step=SC_SUBCORES)
    def _zero_spmem(t):
        pltpu.sync_copy(r0, agg_sh.at[pl.ds(t * ZBLK, ZBLK)])

    @pl.when(s == SC_SUBCORES - 1)
    def _zero_tail():
        pltpu.sync_copy(r0.at[pl.ds(0, ZTAIL)],
                        agg_sh.at[pl.ds(NZB * ZBLK, ZTAIL)])

    plsc.subcore_barrier()

    for k in range(3):
        _iwait(k)
        _gather(k)

    @pl.loop(0, CH_PER_W - 3, step=3)
    def _edges(j):
        for k in range(3):
            _gwait(k)
            _jwait(k)
            _scat(k)
            _load_i(ch0 + j + k + 3, k)
            _swait(k)
            _load_j(ch0 + j + k + 3, k)
            _iwait(k)
            _gather(k)

    for k in range(3):
        _gwait(k)
        _jwait(k)
        _scat(k)
        _swait(k)

    @pl.when(wid < EXTRA_CH)
    def _extra():
        e = NWORK * CH_PER_W + wid
        pltpu.sync_copy(src_hbm.at[pl.ds(e * CHUNK, CHUNK)], i0)
        pltpu.sync_copy(dst_hbm.at[pl.ds(e * CHUNK, CHUNK)], j0)
        pltpu.async_copy(m_hbm.at[i0], r0, sg0).wait()
        pltpu.sync_copy(r0, agg_sh.at[j0], add=True)

    plsc.subcore_barrier()

    @pl.loop(s, NZB, step=SC_SUBCORES)
    def _dump(t):
        pltpu.sync_copy(agg_sh.at[pl.ds(t * ZBLK, ZBLK)],
                        out_hbm.at[c, pl.ds(t * ZBLK, ZBLK)])

    @pl.when(s == SC_SUBCORES - 1)
    def _dump_tail():
        pltpu.sync_copy(agg_sh.at[pl.ds(NZB * ZBLK, ZTAIL)],
                        out_hbm.at[c, pl.ds(NZB * ZBLK, ZTAIL)])


@functools.lru_cache(maxsize=1)
def _get_sc_segment_sum():
    mesh = plsc.VectorSubcoreMesh(core_axis_name="c", subcore_axis_name="s")
    return pl.kernel(
        _sc_segment_sum_body,
        mesh=mesh,
        out_type=jax.ShapeDtypeStruct((SC_CORES, N, H), jnp.float32),
        scratch_types=(
            [pltpu.VMEM((CHUNK,), jnp.int32)] * 6
            + [pltpu.VMEM((CHUNK, H), jnp.float32)] * 3
            + [pltpu.VMEM_SHARED((N, H), jnp.float32)]
            + [pltpu.SemaphoreType.DMA] * 12
        ),
    )


_MM_ROWS = 2000


def _mm_body(h_ref, w_ref, o_ref):
    o_ref[...] = jnp.dot(h_ref[...], w_ref[...],
                         preferred_element_type=jnp.float32)


def _tc_matmul(h, w):
    return pl.pallas_call(
        _mm_body,
        grid=(N // _MM_ROWS,),
        in_specs=[
            pl.BlockSpec((_MM_ROWS, H), lambda i: (i, 0)),
            pl.BlockSpec((H, H), lambda i: (0, 0)),
        ],
        out_specs=pl.BlockSpec((_MM_ROWS, H), lambda i: (i, 0)),
        out_shape=jax.ShapeDtypeStruct((N, H), jnp.float32),
    )(h, w)


_GRU_ROWS = 2000


def _gru_body(parts_ref, h_ref, w_ih_ref, w_hh_ref, b_ih_ref, b_hh_ref, o_ref):
    agg = parts_ref[0] + parts_ref[1]
    h = h_ref[...]
    gi = lax.dot_general(agg, w_ih_ref[...],
                         (((1,), (1,)), ((), ())),
                         preferred_element_type=jnp.float32) + b_ih_ref[...]
    gh = lax.dot_general(h, w_hh_ref[...],
                         (((1,), (1,)), ((), ())),
                         preferred_element_type=jnp.float32) + b_hh_ref[...]
    i_r, i_z, i_n = gi[:, :H], gi[:, H:2 * H], gi[:, 2 * H:]
    h_r, h_z, h_n = gh[:, :H], gh[:, H:2 * H], gh[:, 2 * H:]
    r = jax.nn.sigmoid(i_r + h_r)
    z = jax.nn.sigmoid(i_z + h_z)
    n = jnp.tanh(i_n + r * h_n)
    o_ref[...] = (1.0 - z) * n + z * h


def _tc_gru(parts, h, w_ih, w_hh, b_ih2, b_hh2):
    return pl.pallas_call(
        _gru_body,
        grid=(N // _GRU_ROWS,),
        in_specs=[
            pl.BlockSpec((SC_CORES, _GRU_ROWS, H), lambda i: (0, i, 0)),
            pl.BlockSpec((_GRU_ROWS, H), lambda i: (i, 0)),
            pl.BlockSpec((3 * H, H), lambda i: (0, 0)),
            pl.BlockSpec((3 * H, H), lambda i: (0, 0)),
            pl.BlockSpec((1, 3 * H), lambda i: (0, 0)),
            pl.BlockSpec((1, 3 * H), lambda i: (0, 0)),
        ],
        out_specs=pl.BlockSpec((_GRU_ROWS, H), lambda i: (i, 0)),
        out_shape=jax.ShapeDtypeStruct((N, H), jnp.float32),
    )(parts, h, w_ih, w_hh, b_ih2, b_hh2)


_POOL_ROWS = 1000
_POOL_BLOCKS = N // _POOL_ROWS


def _pool_mlp_body(h_ref, batch_ref, w1_ref, b1_ref, w2_ref, b2_ref,
                   w3_ref, b3_ref, wc_ref, bc_ref, o_ref, pool_acc):
    i = pl.program_id(0)

    @pl.when(i == 0)
    def _():
        pool_acc[...] = jnp.zeros((NG, H), jnp.float32)

    bat = batch_ref[0, 0, :]
    gids = lax.broadcasted_iota(jnp.int32, (NG, _POOL_ROWS), 0)
    onehot = (bat[None, :] == gids).astype(jnp.float32)
    pool_acc[...] += jnp.dot(onehot, h_ref[...],
                             preferred_element_type=jnp.float32)

    @pl.when(i == _POOL_BLOCKS - 1)
    def _():
        g = pool_acc[...]
        f = jax.nn.relu(lax.dot_general(g, w1_ref[...],
                                        (((1,), (1,)), ((), ())),
                                        preferred_element_type=jnp.float32)
                        + b1_ref[...])
        f = jax.nn.relu(lax.dot_general(f, w2_ref[...],
                                        (((1,), (1,)), ((), ())),
                                        preferred_element_type=jnp.float32)
                        + b2_ref[...])
        f = jax.nn.relu(lax.dot_general(f, w3_ref[...],
                                        (((1,), (1,)), ((), ())),
                                        preferred_element_type=jnp.float32)
                        + b3_ref[...])
        o_ref[...] = lax.dot_general(f, wc_ref[...],
                                     (((1,), (1,)), ((), ())),
                                     preferred_element_type=jnp.float32) \
            + bc_ref[...]


def _tc_pool_mlp(h, batch3, W1, b1_2, W2, b2_2, W3, b3_2, Wc, bc_2):
    def full(shape):
        return pl.BlockSpec(shape, lambda i: tuple(0 for _ in shape))
    return pl.pallas_call(
        _pool_mlp_body,
        grid=(_POOL_BLOCKS,),
        in_specs=[
            pl.BlockSpec((_POOL_ROWS, H), lambda i: (i, 0)),
            pl.BlockSpec((1, 1, _POOL_ROWS), lambda i: (i, 0, 0)),
            full((MLP_H, H)),
            full((1, MLP_H)),
            full((H, MLP_H)),
            full((1, H)),
            full((MLP_H, H)),
            full((1, MLP_H)),
            full((NC, MLP_H)),
            full((1, NC)),
        ],
        out_specs=pl.BlockSpec((NG, NC), lambda i: (0, 0)),
        out_shape=jax.ShapeDtypeStruct((NG, NC), jnp.float32),
        scratch_shapes=[pltpu.VMEM((NG, H), jnp.float32)],
    )(h, batch3, W1, b1_2, W2, b2_2, W3, b3_2, Wc, bc_2)


def kernel(x, edge_index, batch, ggnn_weight, w_ih, w_hh, b_ih, b_hh,
           W1, b1, W2, b2, W3, b3, Wc, bc):
    src = edge_index[0]
    dst = edge_index[1]
    b_ih2 = b_ih.reshape(1, 3 * H)
    b_hh2 = b_hh.reshape(1, 3 * H)
    batch3 = batch.reshape(_POOL_BLOCKS, 1, _POOL_ROWS)

    h = x
    for i in range(STEPS):
        m = _tc_matmul(h, ggnn_weight[i])
        parts = _get_sc_segment_sum()(m, src, dst)
        h = _tc_gru(parts, h, w_ih, w_hh, b_ih2, b_hh2)

    return _tc_pool_mlp(h, batch3, W1, b1.reshape(1, MLP_H),
                        W2, b2.reshape(1, H), W3, b3.reshape(1, MLP_H),
                        Wc, bc.reshape(1, NC))


# speedup vs baseline: 13.3017x; 1.0492x over previous
import functools

import jax
import jax.numpy as jnp
from jax import lax
from jax.experimental import pallas as pl
from jax.experimental.pallas import tpu as pltpu
from jax.experimental.pallas import tpu_sc as plsc

N = 10000
E = 320000
H = 128
STEPS = 6
NG = 64
NC = 2
MLP_H = 2 * H

SC_CORES = 2
SC_SUBCORES = 16
NWORK = SC_CORES * SC_SUBCORES
CHUNK = 128
TOT_CH = E // CHUNK
CH_PER_W = TOT_CH // NWORK
EXTRA_CH = TOT_CH - CH_PER_W * NWORK
ZBLK = CHUNK
NZB = N // ZBLK
ZTAIL = N - NZB * ZBLK


def _sc_segment_sum_body(m_hbm, src_hbm, dst_hbm, out_hbm,
                         i0, j0, i1, j1, i2, j2, r0, r1, r2, agg_sh,
                         si0, sj0, si1, sj1, si2, sj2,
                         sg0, sg1, sg2, ss0, ss1, ss2):
    c = lax.axis_index("c")
    s = lax.axis_index("s")
    wid = c * SC_SUBCORES + s
    ch0 = wid * CH_PER_W
    ibufs = (i0, i1, i2)
    jbufs = (j0, j1, j2)
    rbufs = (r0, r1, r2)
    sis = (si0, si1, si2)
    sjs = (sj0, sj1, sj2)
    sgs = (sg0, sg1, sg2)
    sss = (ss0, ss1, ss2)

    def _load_i(j, k):
        pltpu.async_copy(src_hbm.at[pl.ds(j * CHUNK, CHUNK)], ibufs[k], sis[k])

    def _load_j(j, k):
        pltpu.async_copy(dst_hbm.at[pl.ds(j * CHUNK, CHUNK)], jbufs[k], sjs[k])

    def _iwait(k):
        pltpu.make_async_copy(src_hbm.at[pl.ds(0, CHUNK)], ibufs[k],
                              sis[k]).wait()

    def _jwait(k):
        pltpu.make_async_copy(dst_hbm.at[pl.ds(0, CHUNK)], jbufs[k],
                              sjs[k]).wait()

    def _gather(k):
        pltpu.async_copy(m_hbm.at[ibufs[k]], rbufs[k], sgs[k])

    def _gwait(k):
        pltpu.make_async_copy(m_hbm.at[ibufs[k]], rbufs[k], sgs[k]).wait()

    def _scat(k):
        pltpu.async_copy(rbufs[k], agg_sh.at[jbufs[k]], sss[k], add=True)

    def _swait(k):
        pltpu.make_async_copy(rbufs[k], agg_sh.at[jbufs[k]], sss[k]).wait()

    for k in range(3):
        _load_i(ch0 + k, k)
        _load_j(ch0 + k, k)

    @pl.loop(0, ZBLK)
    def _zero_rows(r):
        @pl.loop(0, H, step=16)
        def _zero_lanes(col):
            r0[r, pl.ds(col, 16)] = jnp.zeros((16,), jnp.float32)

    @pl.loop(s, NZB, step=SC_SUBCORES)
    def _zero_spmem(t):
        pltpu.sync_copy(r0, agg_sh.at[pl.ds(t * ZBLK, ZBLK)])

    @pl.when(s == SC_SUBCORES - 1)
    def _zero_tail():
        pltpu.sync_copy(r0.at[pl.ds(0, ZTAIL)],
                        agg_sh.at[pl.ds(NZB * ZBLK, ZTAIL)])

    plsc.subcore_barrier()

    for k in range(3):
        _iwait(k)
        _gather(k)

    @pl.loop(0, CH_PER_W - 3, step=3)
    def _edges(j):
        for k in range(3):
            _gwait(k)
            _jwait(k)
            _scat(k)
            _load_i(ch0 + j + k + 3, k)
            _swait(k)
            _load_j(ch0 + j + k + 3, k)
            _iwait(k)
            _gather(k)

    for k in range(3):
        _gwait(k)
        _jwait(k)
        _scat(k)
        _swait(k)

    @pl.when(wid < EXTRA_CH)
    def _extra():
        e = NWORK * CH_PER_W + wid
        pltpu.sync_copy(src_hbm.at[pl.ds(e * CHUNK, CHUNK)], i0)
        pltpu.sync_copy(dst_hbm.at[pl.ds(e * CHUNK, CHUNK)], j0)
        pltpu.async_copy(m_hbm.at[i0], r0, sg0).wait()
        pltpu.sync_copy(r0, agg_sh.at[j0], add=True)

    plsc.subcore_barrier()

    @pl.loop(s, NZB, step=SC_SUBCORES)
    def _dump(t):
        pltpu.sync_copy(agg_sh.at[pl.ds(t * ZBLK, ZBLK)],
                        out_hbm.at[c, pl.ds(t * ZBLK, ZBLK)])

    @pl.when(s == SC_SUBCORES - 1)
    def _dump_tail():
        pltpu.sync_copy(agg_sh.at[pl.ds(NZB * ZBLK, ZTAIL)],
                        out_hbm.at[c, pl.ds(NZB * ZBLK, ZTAIL)])


@functools.lru_cache(maxsize=1)
def _get_sc_segment_sum():
    mesh = plsc.VectorSubcoreMesh(core_axis_name="c", subcore_axis_name="s")
    return pl.kernel(
        _sc_segment_sum_body,
        mesh=mesh,
        out_type=jax.ShapeDtypeStruct((SC_CORES, N, H), jnp.float32),
        scratch_types=(
            [pltpu.VMEM((CHUNK,), jnp.int32)] * 6
            + [pltpu.VMEM((CHUNK, H), jnp.float32)] * 3
            + [pltpu.VMEM_SHARED((N, H), jnp.float32)]
            + [pltpu.SemaphoreType.DMA] * 12
        ),
    )


_MM_ROWS = 2000


def _mm_body(h_ref, w_ref, o_ref):
    o_ref[...] = jnp.dot(h_ref[...], w_ref[...],
                         preferred_element_type=jnp.float32)


def _tc_matmul(h, w):
    return pl.pallas_call(
        _mm_body,
        grid=(N // _MM_ROWS,),
        in_specs=[
            pl.BlockSpec((_MM_ROWS, H), lambda i: (i, 0)),
            pl.BlockSpec((H, H), lambda i: (0, 0)),
        ],
        out_specs=pl.BlockSpec((_MM_ROWS, H), lambda i: (i, 0)),
        out_shape=jax.ShapeDtypeStruct((N, H), jnp.float32),
    )(h, w)


_GRU_ROWS = 2000


def _gru_rows(parts_ref, h_ref, w_ih_ref, w_hh_ref, b_ih_ref, b_hh_ref):
    agg = parts_ref[0] + parts_ref[1]
    h = h_ref[...]
    gi = lax.dot_general(agg, w_ih_ref[...],
                         (((1,), (1,)), ((), ())),
                         preferred_element_type=jnp.float32) + b_ih_ref[...]
    gh = lax.dot_general(h, w_hh_ref[...],
                         (((1,), (1,)), ((), ())),
                         preferred_element_type=jnp.float32) + b_hh_ref[...]
    i_r, i_z, i_n = gi[:, :H], gi[:, H:2 * H], gi[:, 2 * H:]
    h_r, h_z, h_n = gh[:, :H], gh[:, H:2 * H], gh[:, 2 * H:]
    r = jax.nn.sigmoid(i_r + h_r)
    z = jax.nn.sigmoid(i_z + h_z)
    n = jnp.tanh(i_n + r * h_n)
    return (1.0 - z) * n + z * h


def _gru_mm_body(parts_ref, h_ref, w_ih_ref, w_hh_ref, b_ih_ref, b_hh_ref,
                 w_next_ref, h_out_ref, m_out_ref):
    h_new = _gru_rows(parts_ref, h_ref, w_ih_ref, w_hh_ref, b_ih_ref,
                      b_hh_ref)
    h_out_ref[...] = h_new
    m_out_ref[...] = jnp.dot(h_new, w_next_ref[...],
                             preferred_element_type=jnp.float32)


def _tc_gru_mm(parts, h, w_ih, w_hh, b_ih2, b_hh2, w_next):
    return pl.pallas_call(
        _gru_mm_body,
        grid=(N // _GRU_ROWS,),
        in_specs=[
            pl.BlockSpec((SC_CORES, _GRU_ROWS, H), lambda i: (0, i, 0)),
            pl.BlockSpec((_GRU_ROWS, H), lambda i: (i, 0)),
            pl.BlockSpec((3 * H, H), lambda i: (0, 0)),
            pl.BlockSpec((3 * H, H), lambda i: (0, 0)),
            pl.BlockSpec((1, 3 * H), lambda i: (0, 0)),
            pl.BlockSpec((1, 3 * H), lambda i: (0, 0)),
            pl.BlockSpec((H, H), lambda i: (0, 0)),
        ],
        out_specs=[
            pl.BlockSpec((_GRU_ROWS, H), lambda i: (i, 0)),
            pl.BlockSpec((_GRU_ROWS, H), lambda i: (i, 0)),
        ],
        out_shape=[
            jax.ShapeDtypeStruct((N, H), jnp.float32),
            jax.ShapeDtypeStruct((N, H), jnp.float32),
        ],
    )(parts, h, w_ih, w_hh, b_ih2, b_hh2, w_next)


_POOL_ROWS = 1000
_POOL_BLOCKS = N // _POOL_ROWS


def _pool_mlp_body(parts_ref, h_ref, w_ih_ref, w_hh_ref, b_ih_ref, b_hh_ref,
                   batch_ref, w1_ref, b1_ref, w2_ref, b2_ref,
                   w3_ref, b3_ref, wc_ref, bc_ref, o_ref, pool_acc):
    i = pl.program_id(0)

    @pl.when(i == 0)
    def _():
        pool_acc[...] = jnp.zeros((NG, H), jnp.float32)

    h_new = _gru_rows(parts_ref, h_ref, w_ih_ref, w_hh_ref, b_ih_ref,
                      b_hh_ref)
    bat = batch_ref[0, 0, :]
    gids = lax.broadcasted_iota(jnp.int32, (NG, _POOL_ROWS), 0)
    onehot = (bat[None, :] == gids).astype(jnp.float32)
    pool_acc[...] += jnp.dot(onehot, h_new,
                             preferred_element_type=jnp.float32)

    @pl.when(i == _POOL_BLOCKS - 1)
    def _():
        g = pool_acc[...]
        f = jax.nn.relu(lax.dot_general(g, w1_ref[...],
                                        (((1,), (1,)), ((), ())),
                                        preferred_element_type=jnp.float32)
                        + b1_ref[...])
        f = jax.nn.relu(lax.dot_general(f, w2_ref[...],
                                        (((1,), (1,)), ((), ())),
                                        preferred_element_type=jnp.float32)
                        + b2_ref[...])
        f = jax.nn.relu(lax.dot_general(f, w3_ref[...],
                                        (((1,), (1,)), ((), ())),
                                        preferred_element_type=jnp.float32)
                        + b3_ref[...])
        o_ref[...] = lax.dot_general(f, wc_ref[...],
                                     (((1,), (1,)), ((), ())),
                                     preferred_element_type=jnp.float32) \
            + bc_ref[...]


def _tc_pool_mlp(parts, h, w_ih, w_hh, b_ih2, b_hh2, batch3,
                 W1, b1_2, W2, b2_2, W3, b3_2, Wc, bc_2):
    def full(shape):
        return pl.BlockSpec(shape, lambda i: tuple(0 for _ in shape))
    return pl.pallas_call(
        _pool_mlp_body,
        grid=(_POOL_BLOCKS,),
        in_specs=[
            pl.BlockSpec((SC_CORES, _POOL_ROWS, H), lambda i: (0, i, 0)),
            pl.BlockSpec((_POOL_ROWS, H), lambda i: (i, 0)),
            full((3 * H, H)),
            full((3 * H, H)),
            full((1, 3 * H)),
            full((1, 3 * H)),
            pl.BlockSpec((1, 1, _POOL_ROWS), lambda i: (i, 0, 0)),
            full((MLP_H, H)),
            full((1, MLP_H)),
            full((H, MLP_H)),
            full((1, H)),
            full((MLP_H, H)),
            full((1, MLP_H)),
            full((NC, MLP_H)),
            full((1, NC)),
        ],
        out_specs=pl.BlockSpec((NG, NC), lambda i: (0, 0)),
        out_shape=jax.ShapeDtypeStruct((NG, NC), jnp.float32),
        scratch_shapes=[pltpu.VMEM((NG, H), jnp.float32)],
    )(parts, h, w_ih, w_hh, b_ih2, b_hh2, batch3,
      W1, b1_2, W2, b2_2, W3, b3_2, Wc, bc_2)


def kernel(x, edge_index, batch, ggnn_weight, w_ih, w_hh, b_ih, b_hh,
           W1, b1, W2, b2, W3, b3, Wc, bc):
    src = edge_index[0]
    dst = edge_index[1]
    b_ih2 = b_ih.reshape(1, 3 * H)
    b_hh2 = b_hh.reshape(1, 3 * H)
    batch3 = batch.reshape(_POOL_BLOCKS, 1, _POOL_ROWS)

    sc_seg = _get_sc_segment_sum()
    h = x
    m = _tc_matmul(h, ggnn_weight[0])
    for i in range(STEPS - 1):
        parts = sc_seg(m, src, dst)
        h, m = _tc_gru_mm(parts, h, w_ih, w_hh, b_ih2, b_hh2,
                          ggnn_weight[i + 1])
    parts = sc_seg(m, src, dst)

    return _tc_pool_mlp(parts, h, w_ih, w_hh, b_ih2, b_hh2, batch3,
                        W1, b1.reshape(1, MLP_H),
                        W2, b2.reshape(1, H), W3, b3.reshape(1, MLP_H),
                        Wc, bc.reshape(1, NC))
